# baseline (speedup 1.0000x reference)
import functools

import jax
import jax.numpy as jnp
from jax import lax
from jax.experimental import pallas as pl
from jax.experimental.pallas import tpu as pltpu
from jax.experimental.pallas import tpu_sc as plsc

B, S, D, H, E, DFF = 2, 2048, 768, 12, 64, 3072
DH = D // H
DHP = 128
T = B * S
CAP = T // E
TBLK = 512
QBLK = 256
NTB = T // TBLK
NQB = S // QBLK
DISP_ROWS = E * CAP + CAP

_SC_TOK = 128


def _ln_block(xb, g, b):
    m = jnp.mean(xb, axis=1, keepdims=True)
    c = xb - m
    v = jnp.mean(c * c, axis=1, keepdims=True)
    return c * jax.lax.rsqrt(v + 1e-5) * g + b


def _qkv_kernel(x_ref, g_ref, b_ref, w_ref, wb_ref, qkv_ref):
    h = _ln_block(x_ref[...], g_ref[...], b_ref[...]).astype(jnp.bfloat16)
    qkv = lax.dot_general(h, w_ref[...], (((1,), (1,)), ((), ())),
                          preferred_element_type=jnp.float32)
    qkv_ref[...] = (qkv + wb_ref[...]).astype(jnp.bfloat16)


def _qkv_call(x2d, ln1_g, ln1_b, in_proj_w, in_proj_b, interpret=False):
    w_pad = jnp.pad(in_proj_w.reshape(3 * H, DH, D), ((0, 0), (0, DHP - DH), (0, 0))
                    ).reshape(3 * H * DHP, D).astype(jnp.bfloat16)
    b_pad = jnp.pad(in_proj_b.reshape(3 * H, DH), ((0, 0), (0, DHP - DH))
                    ).reshape(1, 3 * H * DHP)
    return pl.pallas_call(
        _qkv_kernel,
        grid=(NTB,),
        in_specs=[
            pl.BlockSpec((TBLK, D), lambda i: (i, 0)),
            pl.BlockSpec((1, D), lambda i: (0, 0)),
            pl.BlockSpec((1, D), lambda i: (0, 0)),
            pl.BlockSpec((3 * H * DHP, D), lambda i: (0, 0)),
            pl.BlockSpec((1, 3 * H * DHP), lambda i: (0, 0)),
        ],
        out_specs=pl.BlockSpec((TBLK, 3 * H * DHP), lambda i: (i, 0)),
        out_shape=jax.ShapeDtypeStruct((T, 3 * H * DHP), jnp.bfloat16),
        interpret=interpret,
    )(x2d, ln1_g.reshape(1, D), ln1_b.reshape(1, D), w_pad, b_pad)


def _attn_kernel(q_ref, k_ref, v_ref, o_ref):
    s = lax.dot_general(q_ref[...], k_ref[...], (((1,), (1,)), ((), ())),
                        preferred_element_type=jnp.float32)
    s = s * (1.0 / 8.0)
    m = jnp.max(s, axis=1, keepdims=True)
    p = jnp.exp(s - m)
    p = (p / jnp.sum(p, axis=1, keepdims=True)).astype(jnp.bfloat16)
    o_ref[...] = jnp.dot(p, v_ref[...], preferred_element_type=jnp.float32
                         ).astype(jnp.bfloat16)


def _attn_call(qkv, interpret=False):
    return pl.pallas_call(
        _attn_kernel,
        grid=(B * H, NQB),
        in_specs=[
            pl.BlockSpec((QBLK, DHP), lambda bh, qi: ((bh // H) * NQB + qi, bh % H)),
            pl.BlockSpec((S, DHP), lambda bh, qi: (bh // H, H + bh % H)),
            pl.BlockSpec((S, DHP), lambda bh, qi: (bh // H, 2 * H + bh % H)),
        ],
        out_specs=pl.BlockSpec((QBLK, DHP), lambda bh, qi: ((bh // H) * NQB + qi, bh % H)),
        out_shape=jax.ShapeDtypeStruct((T, H * DHP), jnp.bfloat16),
        interpret=interpret,
    )(qkv, qkv, qkv)


def _post_kernel(ao_ref, x_ref, wo_ref, bo_ref, g2_ref, b2_ref, gw_ref,
                 x2_ref, tok_ref, dsts_ref, dstg_ref, gate_ref, cnt_ref):
    i = pl.program_id(0)
    ao = lax.dot_general(ao_ref[...], wo_ref[...], (((1,), (1,)), ((), ())),
                         preferred_element_type=jnp.float32)
    x2 = x_ref[...] + ao + bo_ref[...]
    x2_ref[...] = x2
    h = _ln_block(x2, g2_ref[...], b2_ref[...])
    tok_ref[...] = h
    logits = jnp.dot(h, gw_ref[...], preferred_element_type=jnp.float32)

    m = jnp.max(logits, axis=1, keepdims=True)
    ex = jnp.exp(logits - m)
    sm = ex / jnp.sum(ex, axis=1, keepdims=True)

    lane = lax.broadcasted_iota(jnp.int32, (TBLK, E), 1)
    idx = jnp.min(jnp.where(logits == m, lane, E), axis=1, keepdims=True)
    onehot = (lane == idx).astype(jnp.float32)
    gate_raw = jnp.sum(sm * onehot, axis=1, keepdims=True)

    @pl.when(i == 0)
    def _():
        cnt_ref[...] = jnp.zeros((1, E), jnp.float32)

    base = cnt_ref[...]
    r = lax.broadcasted_iota(jnp.int32, (TBLK, TBLK), 0)
    c = lax.broadcasted_iota(jnp.int32, (TBLK, TBLK), 1)
    tril = (r > c).astype(jnp.float32)
    excl = jnp.dot(tril, onehot, preferred_element_type=jnp.float32)
    loc = jnp.sum((excl + base) * onehot, axis=1, keepdims=True).astype(jnp.int32)
    cnt_ref[...] = base + jnp.sum(onehot, axis=0, keepdims=True)

    keep = loc < CAP
    loc_c = jnp.minimum(loc, CAP - 1)
    dest_g = idx * CAP + loc_c
    dsts_ref[...] = jnp.where(keep, dest_g, E * CAP)
    dstg_ref[...] = dest_g
    gate_ref[...] = jnp.where(keep, gate_raw, 0.0)


def _post_call(ao, x2d, out_proj_w, out_proj_b, ln2_g, ln2_b, gate_w,
               interpret=False):
    blk2 = pl.BlockSpec((TBLK, 1), lambda i: (i, 0))
    wo_pad = jnp.pad(out_proj_w.reshape(D, H, DH), ((0, 0), (0, 0), (0, DHP - DH))
                     ).reshape(D, H * DHP).astype(jnp.bfloat16)
    return pl.pallas_call(
        _post_kernel,
        grid=(NTB,),
        in_specs=[
            pl.BlockSpec((TBLK, H * DHP), lambda i: (i, 0)),
            pl.BlockSpec((TBLK, D), lambda i: (i, 0)),
            pl.BlockSpec((D, H * DHP), lambda i: (0, 0)),
            pl.BlockSpec((1, D), lambda i: (0, 0)),
            pl.BlockSpec((1, D), lambda i: (0, 0)),
            pl.BlockSpec((1, D), lambda i: (0, 0)),
            pl.BlockSpec((D, E), lambda i: (0, 0)),
        ],
        out_specs=(
            pl.BlockSpec((TBLK, D), lambda i: (i, 0)),
            pl.BlockSpec((TBLK, D), lambda i: (i, 0)),
            blk2, blk2, blk2,
        ),
        out_shape=(
            jax.ShapeDtypeStruct((T, D), jnp.float32),
            jax.ShapeDtypeStruct((T, D), jnp.float32),
            jax.ShapeDtypeStruct((T, 1), jnp.int32),
            jax.ShapeDtypeStruct((T, 1), jnp.int32),
            jax.ShapeDtypeStruct((T, 1), jnp.float32),
        ),
        scratch_shapes=[pltpu.VMEM((1, E), jnp.float32)],
        interpret=interpret,
    )(ao, x2d, wo_pad, out_proj_b.reshape(1, D), ln2_g.reshape(1, D),
      ln2_b.reshape(1, D), gate_w)


def _sc_worker_id():
    return lax.axis_index("s") * 2 + lax.axis_index("c")


def _sc_dispatch_body(tok_hbm, dst_hbm, out_hbm, idx_v, rows_v, sem):
    base = _sc_worker_id() * _SC_TOK
    pltpu.sync_copy(dst_hbm.at[pl.ds(base, _SC_TOK)], idx_v)
    pltpu.sync_copy(tok_hbm.at[pl.ds(base, _SC_TOK)], rows_v)
    pltpu.async_copy(rows_v, out_hbm.at[idx_v], sem).wait()


def _sc_dispatch(tokens, dest_s):
    mesh = plsc.VectorSubcoreMesh(core_axis_name="c", subcore_axis_name="s")
    return pl.kernel(
        _sc_dispatch_body,
        out_type=jax.ShapeDtypeStruct((DISP_ROWS, D), jnp.float32),
        mesh=mesh,
        scratch_types=[
            pltpu.VMEM((_SC_TOK,), jnp.int32),
            pltpu.VMEM((_SC_TOK, D), jnp.float32),
            pltpu.SemaphoreType.DMA,
        ],
    )(tokens, dest_s)


def _sc_combine_body(eout_hbm, dst_hbm, out_hbm, idx_v, rows_v, sem):
    base = _sc_worker_id() * _SC_TOK
    pltpu.sync_copy(dst_hbm.at[pl.ds(base, _SC_TOK)], idx_v)
    pltpu.async_copy(eout_hbm.at[idx_v], rows_v, sem).wait()
    pltpu.sync_copy(rows_v, out_hbm.at[pl.ds(base, _SC_TOK)])


def _sc_combine(eout2d, dest_g):
    mesh = plsc.VectorSubcoreMesh(core_axis_name="c", subcore_axis_name="s")
    return pl.kernel(
        _sc_combine_body,
        out_type=jax.ShapeDtypeStruct((T, D), jnp.float32),
        mesh=mesh,
        scratch_types=[
            pltpu.VMEM((_SC_TOK,), jnp.int32),
            pltpu.VMEM((_SC_TOK, D), jnp.float32),
            pltpu.SemaphoreType.DMA,
        ],
    )(eout2d, dest_g)


NFS = 4
DFH = DFF // NFS


def _ffn_kernel(disp_ref, w1_ref, b1_ref, w2_ref, b2_ref, out_ref):
    j = pl.program_id(1)
    h1 = jnp.dot(disp_ref[...], w1_ref[0], preferred_element_type=jnp.float32)
    h1 = h1 + b1_ref[0]
    h1 = 0.5 * h1 * (1.0 + lax.erf(h1 * 0.7071067811865476))
    part = jnp.dot(h1, w2_ref[0], preferred_element_type=jnp.float32)

    @pl.when(j == 0)
    def _():
        out_ref[0] = part + b2_ref[0]

    @pl.when(j > 0)
    def _():
        out_ref[0] += part


def _ffn_call(disp, w1, b1, w2, b2, interpret=False):
    return pl.pallas_call(
        _ffn_kernel,
        grid=(E, NFS),
        in_specs=[
            pl.BlockSpec((CAP, D), lambda e, j: (e, 0)),
            pl.BlockSpec((1, D, DFH), lambda e, j: (e, 0, j)),
            pl.BlockSpec((1, 1, DFH), lambda e, j: (e, 0, j)),
            pl.BlockSpec((1, DFH, D), lambda e, j: (e, j, 0)),
            pl.BlockSpec((1, 1, D), lambda e, j: (e, 0, 0)),
        ],
        out_specs=pl.BlockSpec((1, CAP, D), lambda e, j: (e, 0, 0)),
        out_shape=jax.ShapeDtypeStruct((E, CAP, D), jnp.float32),
        interpret=interpret,
    )(disp, w1, b1.reshape(E, 1, DFF), w2, b2.reshape(E, 1, D))


def _epi_kernel(x2_ref, gat_ref, gv_ref, o_ref):
    o_ref[...] = x2_ref[...] + gat_ref[...] * gv_ref[...]


def _epi_call(x2, gathered, gate_val, interpret=False):
    return pl.pallas_call(
        _epi_kernel,
        grid=(NTB,),
        in_specs=[
            pl.BlockSpec((TBLK, D), lambda i: (i, 0)),
            pl.BlockSpec((TBLK, D), lambda i: (i, 0)),
            pl.BlockSpec((TBLK, 1), lambda i: (i, 0)),
        ],
        out_specs=pl.BlockSpec((TBLK, D), lambda i: (i, 0)),
        out_shape=jax.ShapeDtypeStruct((T, D), jnp.float32),
        interpret=interpret,
    )(x2, gathered, gate_val)


def kernel(x, ln1_g, ln1_b, in_proj_w, in_proj_b, out_proj_w, out_proj_b,
           ln2_g, ln2_b, gate_w, w1, b1, w2, b2):
    x2d = x.reshape(T, D)
    qkv = _qkv_call(x2d, ln1_g, ln1_b, in_proj_w, in_proj_b)
    ao = _attn_call(qkv)
    x2, tokens, dest_s, dest_g, gate_val = _post_call(
        ao, x2d, out_proj_w, out_proj_b, ln2_g, ln2_b, gate_w)
    disp = _sc_dispatch(tokens, dest_s.reshape(T))
    eout = _ffn_call(disp, w1, b1, w2, b2)
    gathered = _sc_combine(eout.reshape(T, D), dest_g.reshape(T))
    out = _epi_call(x2, gathered, gate_val)
    return out.reshape(B, S, D)


# speedup vs baseline: 1.0796x; 1.0796x over previous
import functools

import jax
import jax.numpy as jnp
from jax import lax
from jax.experimental import pallas as pl
from jax.experimental.pallas import tpu as pltpu
from jax.experimental.pallas import tpu_sc as plsc

B, S, D, H, E, DFF = 2, 2048, 768, 12, 64, 3072
DH = D // H
DHP = 128
T = B * S
CAP = T // E
TBLK = 512
QBLK = 256
NTB = T // TBLK
NQB = S // QBLK
DISP_ROWS = E * CAP + CAP

_SC_TOK = 128


def _ln_block(xb, g, b):
    m = jnp.mean(xb, axis=1, keepdims=True)
    c = xb - m
    v = jnp.mean(c * c, axis=1, keepdims=True)
    return c * jax.lax.rsqrt(v + 1e-5) * g + b


def _qkv_kernel(x_ref, g_ref, b_ref, w_ref, wb_ref, qkv_ref):
    h = _ln_block(x_ref[...], g_ref[...], b_ref[...]).astype(jnp.bfloat16)
    qkv = lax.dot_general(h, w_ref[...], (((1,), (1,)), ((), ())),
                          preferred_element_type=jnp.float32)
    qkv_ref[...] = (qkv + wb_ref[...]).astype(jnp.bfloat16)


def _qkv_call(x2d, ln1_g, ln1_b, in_proj_w, in_proj_b, interpret=False):
    w_pad = jnp.pad(in_proj_w.reshape(3 * H, DH, D), ((0, 0), (0, DHP - DH), (0, 0))
                    ).reshape(3 * H * DHP, D).astype(jnp.bfloat16)
    b_pad = jnp.pad(in_proj_b.reshape(3 * H, DH), ((0, 0), (0, DHP - DH))
                    ).reshape(1, 3 * H * DHP)
    return pl.pallas_call(
        _qkv_kernel,
        grid=(NTB,),
        in_specs=[
            pl.BlockSpec((TBLK, D), lambda i: (i, 0)),
            pl.BlockSpec((1, D), lambda i: (0, 0)),
            pl.BlockSpec((1, D), lambda i: (0, 0)),
            pl.BlockSpec((3 * H * DHP, D), lambda i: (0, 0)),
            pl.BlockSpec((1, 3 * H * DHP), lambda i: (0, 0)),
        ],
        out_specs=pl.BlockSpec((TBLK, 3 * H * DHP), lambda i: (i, 0)),
        out_shape=jax.ShapeDtypeStruct((T, 3 * H * DHP), jnp.bfloat16),
        interpret=interpret,
    )(x2d, ln1_g.reshape(1, D), ln1_b.reshape(1, D), w_pad, b_pad)


def _attn_kernel(q_ref, k_ref, v_ref, o_ref):
    s = lax.dot_general(q_ref[...], k_ref[...], (((1,), (1,)), ((), ())),
                        preferred_element_type=jnp.float32)
    s = s * (1.0 / 8.0)
    m = jnp.max(s, axis=1, keepdims=True)
    p = jnp.exp(s - m)
    p = (p / jnp.sum(p, axis=1, keepdims=True)).astype(jnp.bfloat16)
    o_ref[...] = jnp.dot(p, v_ref[...], preferred_element_type=jnp.float32
                         ).astype(jnp.bfloat16)


def _attn_call(qkv, interpret=False):
    return pl.pallas_call(
        _attn_kernel,
        grid=(B * H, NQB),
        in_specs=[
            pl.BlockSpec((QBLK, DHP), lambda bh, qi: ((bh // H) * NQB + qi, bh % H)),
            pl.BlockSpec((S, DHP), lambda bh, qi: (bh // H, H + bh % H)),
            pl.BlockSpec((S, DHP), lambda bh, qi: (bh // H, 2 * H + bh % H)),
        ],
        out_specs=pl.BlockSpec((QBLK, DHP), lambda bh, qi: ((bh // H) * NQB + qi, bh % H)),
        out_shape=jax.ShapeDtypeStruct((T, H * DHP), jnp.bfloat16),
        interpret=interpret,
    )(qkv, qkv, qkv)


def _post_kernel(ao_ref, x_ref, wo_ref, bo_ref, g2_ref, b2_ref, gw_ref,
                 x2_ref, tok_ref, dsts_ref, dstg_ref, gate_ref, cnt_ref):
    i = pl.program_id(0)
    ao = lax.dot_general(ao_ref[...], wo_ref[...], (((1,), (1,)), ((), ())),
                         preferred_element_type=jnp.float32)
    x2 = x_ref[...] + ao + bo_ref[...]
    x2_ref[...] = x2
    h = _ln_block(x2, g2_ref[...], b2_ref[...])
    tok_ref[...] = h
    logits = jnp.dot(h, gw_ref[...], preferred_element_type=jnp.float32)

    m = jnp.max(logits, axis=1, keepdims=True)
    ex = jnp.exp(logits - m)
    sm = ex / jnp.sum(ex, axis=1, keepdims=True)

    lane = lax.broadcasted_iota(jnp.int32, (TBLK, E), 1)
    idx = jnp.min(jnp.where(logits == m, lane, E), axis=1, keepdims=True)
    onehot = (lane == idx).astype(jnp.float32)
    gate_raw = jnp.sum(sm * onehot, axis=1, keepdims=True)

    @pl.when(i == 0)
    def _():
        cnt_ref[...] = jnp.zeros((1, E), jnp.float32)

    base = cnt_ref[...]
    r = lax.broadcasted_iota(jnp.int32, (TBLK, TBLK), 0)
    c = lax.broadcasted_iota(jnp.int32, (TBLK, TBLK), 1)
    tril = (r > c).astype(jnp.float32)
    excl = jnp.dot(tril, onehot, preferred_element_type=jnp.float32)
    loc = jnp.sum((excl + base) * onehot, axis=1, keepdims=True).astype(jnp.int32)
    cnt_ref[...] = base + jnp.sum(onehot, axis=0, keepdims=True)

    keep = loc < CAP
    loc_c = jnp.minimum(loc, CAP - 1)
    dest_g = idx * CAP + loc_c
    dsts_ref[...] = jnp.where(keep, dest_g, E * CAP)
    dstg_ref[...] = dest_g
    gate_ref[...] = jnp.where(keep, gate_raw, 0.0)


def _post_call(ao, x2d, out_proj_w, out_proj_b, ln2_g, ln2_b, gate_w,
               interpret=False):
    blk2 = pl.BlockSpec((TBLK, 1), lambda i: (i, 0))
    wo_pad = jnp.pad(out_proj_w.reshape(D, H, DH), ((0, 0), (0, 0), (0, DHP - DH))
                     ).reshape(D, H * DHP).astype(jnp.bfloat16)
    return pl.pallas_call(
        _post_kernel,
        grid=(NTB,),
        in_specs=[
            pl.BlockSpec((TBLK, H * DHP), lambda i: (i, 0)),
            pl.BlockSpec((TBLK, D), lambda i: (i, 0)),
            pl.BlockSpec((D, H * DHP), lambda i: (0, 0)),
            pl.BlockSpec((1, D), lambda i: (0, 0)),
            pl.BlockSpec((1, D), lambda i: (0, 0)),
            pl.BlockSpec((1, D), lambda i: (0, 0)),
            pl.BlockSpec((D, E), lambda i: (0, 0)),
        ],
        out_specs=(
            pl.BlockSpec((TBLK, D), lambda i: (i, 0)),
            pl.BlockSpec((TBLK, D), lambda i: (i, 0)),
            blk2, blk2, blk2,
        ),
        out_shape=(
            jax.ShapeDtypeStruct((T, D), jnp.float32),
            jax.ShapeDtypeStruct((T, D), jnp.float32),
            jax.ShapeDtypeStruct((T, 1), jnp.int32),
            jax.ShapeDtypeStruct((T, 1), jnp.int32),
            jax.ShapeDtypeStruct((T, 1), jnp.float32),
        ),
        scratch_shapes=[pltpu.VMEM((1, E), jnp.float32)],
        interpret=interpret,
    )(ao, x2d, wo_pad, out_proj_b.reshape(1, D), ln2_g.reshape(1, D),
      ln2_b.reshape(1, D), gate_w)


def _sc_worker_id():
    return lax.axis_index("s") * 2 + lax.axis_index("c")


def _sc_dispatch_body(tok_hbm, dst_hbm, out_hbm, idx_v, rows_v, sem):
    base = _sc_worker_id() * _SC_TOK
    pltpu.sync_copy(dst_hbm.at[pl.ds(base, _SC_TOK)], idx_v)
    pltpu.sync_copy(tok_hbm.at[pl.ds(base, _SC_TOK)], rows_v)
    pltpu.async_copy(rows_v, out_hbm.at[idx_v], sem).wait()


def _sc_dispatch(tokens, dest_s):
    mesh = plsc.VectorSubcoreMesh(core_axis_name="c", subcore_axis_name="s")
    return pl.kernel(
        _sc_dispatch_body,
        out_type=jax.ShapeDtypeStruct((DISP_ROWS, D), jnp.float32),
        mesh=mesh,
        scratch_types=[
            pltpu.VMEM((_SC_TOK,), jnp.int32),
            pltpu.VMEM((_SC_TOK, D), jnp.float32),
            pltpu.SemaphoreType.DMA,
        ],
    )(tokens, dest_s)


def _sc_combine_body(eout_hbm, dst_hbm, out_hbm, idx_v, rows_v, sem):
    base = _sc_worker_id() * _SC_TOK
    pltpu.sync_copy(dst_hbm.at[pl.ds(base, _SC_TOK)], idx_v)
    pltpu.async_copy(eout_hbm.at[idx_v], rows_v, sem).wait()
    pltpu.sync_copy(rows_v, out_hbm.at[pl.ds(base, _SC_TOK)])


def _sc_combine(eout2d, dest_g):
    mesh = plsc.VectorSubcoreMesh(core_axis_name="c", subcore_axis_name="s")
    return pl.kernel(
        _sc_combine_body,
        out_type=jax.ShapeDtypeStruct((T, D), jnp.float32),
        mesh=mesh,
        scratch_types=[
            pltpu.VMEM((_SC_TOK,), jnp.int32),
            pltpu.VMEM((_SC_TOK, D), jnp.float32),
            pltpu.SemaphoreType.DMA,
        ],
    )(eout2d, dest_g)


NFS = 2
DFH = DFF // NFS


def _ffn_kernel(disp_ref, w1_ref, b1_ref, w2_ref, b2_ref, out_ref):
    j = pl.program_id(1)
    h1 = jnp.dot(disp_ref[...], w1_ref[0], preferred_element_type=jnp.float32)
    h1 = h1 + b1_ref[0]
    h1 = 0.5 * h1 * (1.0 + lax.erf(h1 * 0.7071067811865476))
    part = jnp.dot(h1, w2_ref[0], preferred_element_type=jnp.float32)

    @pl.when(j == 0)
    def _():
        out_ref[0] = part + b2_ref[0]

    @pl.when(j > 0)
    def _():
        out_ref[0] += part


def _ffn_call(disp, w1, b1, w2, b2, interpret=False):
    return pl.pallas_call(
        _ffn_kernel,
        grid=(E, NFS),
        in_specs=[
            pl.BlockSpec((CAP, D), lambda e, j: (e, 0)),
            pl.BlockSpec((1, D, DFH), lambda e, j: (e, 0, j)),
            pl.BlockSpec((1, 1, DFH), lambda e, j: (e, 0, j)),
            pl.BlockSpec((1, DFH, D), lambda e, j: (e, j, 0)),
            pl.BlockSpec((1, 1, D), lambda e, j: (e, 0, 0)),
        ],
        out_specs=pl.BlockSpec((1, CAP, D), lambda e, j: (e, 0, 0)),
        out_shape=jax.ShapeDtypeStruct((E, CAP, D), jnp.float32),
        interpret=interpret,
    )(disp, w1, b1.reshape(E, 1, DFF), w2, b2.reshape(E, 1, D))


def _epi_kernel(x2_ref, gat_ref, gv_ref, o_ref):
    o_ref[...] = x2_ref[...] + gat_ref[...] * gv_ref[...]


def _epi_call(x2, gathered, gate_val, interpret=False):
    return pl.pallas_call(
        _epi_kernel,
        grid=(NTB,),
        in_specs=[
            pl.BlockSpec((TBLK, D), lambda i: (i, 0)),
            pl.BlockSpec((TBLK, D), lambda i: (i, 0)),
            pl.BlockSpec((TBLK, 1), lambda i: (i, 0)),
        ],
        out_specs=pl.BlockSpec((TBLK, D), lambda i: (i, 0)),
        out_shape=jax.ShapeDtypeStruct((T, D), jnp.float32),
        interpret=interpret,
    )(x2, gathered, gate_val)


def kernel(x, ln1_g, ln1_b, in_proj_w, in_proj_b, out_proj_w, out_proj_b,
           ln2_g, ln2_b, gate_w, w1, b1, w2, b2):
    x2d = x.reshape(T, D)
    qkv = _qkv_call(x2d, ln1_g, ln1_b, in_proj_w, in_proj_b)
    ao = _attn_call(qkv)
    x2, tokens, dest_s, dest_g, gate_val = _post_call(
        ao, x2d, out_proj_w, out_proj_b, ln2_g, ln2_b, gate_w)
    disp = _sc_dispatch(tokens, dest_s.reshape(T))
    eout = _ffn_call(disp, w1, b1, w2, b2)
    gathered = _sc_combine(eout.reshape(T, D), dest_g.reshape(T))
    out = _epi_call(x2, gathered, gate_val)
    return out.reshape(B, S, D)


# speedup vs baseline: 1.1084x; 1.0267x over previous
import functools

import jax
import jax.numpy as jnp
from jax import lax
from jax.experimental import pallas as pl
from jax.experimental.pallas import tpu as pltpu
from jax.experimental.pallas import tpu_sc as plsc

B, S, D, H, E, DFF = 2, 2048, 768, 12, 64, 3072
DH = D // H
DHP = 128
T = B * S
CAP = T // E
TBLK = 512
QBLK = 256
NTB = T // TBLK
NQB = S // QBLK
DISP_ROWS = E * CAP + CAP

_SC_TOK = 128


def _ln_block(xb, g, b):
    m = jnp.mean(xb, axis=1, keepdims=True)
    c = xb - m
    v = jnp.mean(c * c, axis=1, keepdims=True)
    return c * jax.lax.rsqrt(v + 1e-5) * g + b


def _qkv_kernel(x_ref, g_ref, b_ref, w_ref, wb_ref, qkv_ref):
    h = _ln_block(x_ref[...], g_ref[...], b_ref[...]).astype(jnp.bfloat16)
    qkv = lax.dot_general(h, w_ref[...], (((1,), (1,)), ((), ())),
                          preferred_element_type=jnp.float32)
    qkv_ref[...] = (qkv + wb_ref[...]).astype(jnp.bfloat16)


def _qkv_call(x2d, ln1_g, ln1_b, in_proj_w, in_proj_b, interpret=False):
    qscale = jnp.concatenate([
        jnp.full((H * DHP,), 0.125 * 1.4426950408889634, jnp.float32),
        jnp.ones((2 * H * DHP,), jnp.float32)])
    w_pad = jnp.pad(in_proj_w.reshape(3 * H, DH, D), ((0, 0), (0, DHP - DH), (0, 0))
                    ).reshape(3 * H * DHP, D)
    w_pad = (w_pad * qscale[:, None]).astype(jnp.bfloat16)
    b_pad = jnp.pad(in_proj_b.reshape(3 * H, DH), ((0, 0), (0, DHP - DH))
                    ).reshape(1, 3 * H * DHP) * qscale[None, :]
    return pl.pallas_call(
        _qkv_kernel,
        grid=(NTB,),
        in_specs=[
            pl.BlockSpec((TBLK, D), lambda i: (i, 0)),
            pl.BlockSpec((1, D), lambda i: (0, 0)),
            pl.BlockSpec((1, D), lambda i: (0, 0)),
            pl.BlockSpec((3 * H * DHP, D), lambda i: (0, 0)),
            pl.BlockSpec((1, 3 * H * DHP), lambda i: (0, 0)),
        ],
        out_specs=pl.BlockSpec((TBLK, 3 * H * DHP), lambda i: (i, 0)),
        out_shape=jax.ShapeDtypeStruct((T, 3 * H * DHP), jnp.bfloat16),
        interpret=interpret,
    )(x2d, ln1_g.reshape(1, D), ln1_b.reshape(1, D), w_pad, b_pad)


def _attn_kernel(q_ref, k_ref, v_ref, o_ref):
    s = lax.dot_general(q_ref[...], k_ref[...], (((1,), (1,)), ((), ())),
                        preferred_element_type=jnp.float32)
    m = jnp.max(s, axis=1, keepdims=True)
    p = jnp.exp2(s - m)
    r = 1.0 / jnp.sum(p, axis=1, keepdims=True)
    o = jnp.dot(p.astype(jnp.bfloat16), v_ref[...],
                preferred_element_type=jnp.float32)
    o_ref[...] = (o * r).astype(jnp.bfloat16)


def _attn_call(qkv, interpret=False):
    return pl.pallas_call(
        _attn_kernel,
        grid=(B * H, NQB),
        in_specs=[
            pl.BlockSpec((QBLK, DHP), lambda bh, qi: ((bh // H) * NQB + qi, bh % H)),
            pl.BlockSpec((S, DHP), lambda bh, qi: (bh // H, H + bh % H)),
            pl.BlockSpec((S, DHP), lambda bh, qi: (bh // H, 2 * H + bh % H)),
        ],
        out_specs=pl.BlockSpec((QBLK, DHP), lambda bh, qi: ((bh // H) * NQB + qi, bh % H)),
        out_shape=jax.ShapeDtypeStruct((T, H * DHP), jnp.bfloat16),
        interpret=interpret,
    )(qkv, qkv, qkv)


def _post_kernel(ao_ref, x_ref, wo_ref, bo_ref, g2_ref, b2_ref, gw_ref,
                 x2_ref, tok_ref, dsts_ref, dstg_ref, gate_ref, cnt_ref):
    i = pl.program_id(0)
    ao = lax.dot_general(ao_ref[...], wo_ref[...], (((1,), (1,)), ((), ())),
                         preferred_element_type=jnp.float32)
    x2 = x_ref[...] + ao + bo_ref[...]
    x2_ref[...] = x2
    h = _ln_block(x2, g2_ref[...], b2_ref[...])
    tok_ref[...] = h
    logits = jnp.dot(h, gw_ref[...], preferred_element_type=jnp.float32)

    m = jnp.max(logits, axis=1, keepdims=True)
    ex = jnp.exp(logits - m)
    sm = ex / jnp.sum(ex, axis=1, keepdims=True)

    lane = lax.broadcasted_iota(jnp.int32, (TBLK, E), 1)
    idx = jnp.min(jnp.where(logits == m, lane, E), axis=1, keepdims=True)
    onehot = (lane == idx).astype(jnp.float32)
    gate_raw = jnp.sum(sm * onehot, axis=1, keepdims=True)

    @pl.when(i == 0)
    def _():
        cnt_ref[...] = jnp.zeros((1, E), jnp.float32)

    base = cnt_ref[...]
    r = lax.broadcasted_iota(jnp.int32, (TBLK, TBLK), 0)
    c = lax.broadcasted_iota(jnp.int32, (TBLK, TBLK), 1)
    tril = (r > c).astype(jnp.float32)
    excl = jnp.dot(tril, onehot, preferred_element_type=jnp.float32)
    loc = jnp.sum((excl + base) * onehot, axis=1, keepdims=True).astype(jnp.int32)
    cnt_ref[...] = base + jnp.sum(onehot, axis=0, keepdims=True)

    keep = loc < CAP
    loc_c = jnp.minimum(loc, CAP - 1)
    dest_g = idx * CAP + loc_c
    dsts_ref[...] = jnp.where(keep, dest_g, E * CAP)
    dstg_ref[...] = dest_g
    gate_ref[...] = jnp.where(keep, gate_raw, 0.0)


def _post_call(ao, x2d, out_proj_w, out_proj_b, ln2_g, ln2_b, gate_w,
               interpret=False):
    blk2 = pl.BlockSpec((TBLK, 1), lambda i: (i, 0))
    wo_pad = jnp.pad(out_proj_w.reshape(D, H, DH), ((0, 0), (0, 0), (0, DHP - DH))
                     ).reshape(D, H * DHP).astype(jnp.bfloat16)
    return pl.pallas_call(
        _post_kernel,
        grid=(NTB,),
        in_specs=[
            pl.BlockSpec((TBLK, H * DHP), lambda i: (i, 0)),
            pl.BlockSpec((TBLK, D), lambda i: (i, 0)),
            pl.BlockSpec((D, H * DHP), lambda i: (0, 0)),
            pl.BlockSpec((1, D), lambda i: (0, 0)),
            pl.BlockSpec((1, D), lambda i: (0, 0)),
            pl.BlockSpec((1, D), lambda i: (0, 0)),
            pl.BlockSpec((D, E), lambda i: (0, 0)),
        ],
        out_specs=(
            pl.BlockSpec((TBLK, D), lambda i: (i, 0)),
            pl.BlockSpec((TBLK, D), lambda i: (i, 0)),
            blk2, blk2, blk2,
        ),
        out_shape=(
            jax.ShapeDtypeStruct((T, D), jnp.float32),
            jax.ShapeDtypeStruct((T, D), jnp.float32),
            jax.ShapeDtypeStruct((T, 1), jnp.int32),
            jax.ShapeDtypeStruct((T, 1), jnp.int32),
            jax.ShapeDtypeStruct((T, 1), jnp.float32),
        ),
        scratch_shapes=[pltpu.VMEM((1, E), jnp.float32)],
        interpret=interpret,
    )(ao, x2d, wo_pad, out_proj_b.reshape(1, D), ln2_g.reshape(1, D),
      ln2_b.reshape(1, D), gate_w)


def _sc_worker_id():
    return lax.axis_index("s") * 2 + lax.axis_index("c")


def _sc_dispatch_body(tok_hbm, dst_hbm, out_hbm, idx_v, rows_v, sem):
    base = _sc_worker_id() * _SC_TOK
    pltpu.sync_copy(dst_hbm.at[pl.ds(base, _SC_TOK)], idx_v)
    pltpu.sync_copy(tok_hbm.at[pl.ds(base, _SC_TOK)], rows_v)
    pltpu.async_copy(rows_v, out_hbm.at[idx_v], sem).wait()


def _sc_dispatch(tokens, dest_s):
    mesh = plsc.VectorSubcoreMesh(core_axis_name="c", subcore_axis_name="s")
    return pl.kernel(
        _sc_dispatch_body,
        out_type=jax.ShapeDtypeStruct((DISP_ROWS, D), jnp.float32),
        mesh=mesh,
        scratch_types=[
            pltpu.VMEM((_SC_TOK,), jnp.int32),
            pltpu.VMEM((_SC_TOK, D), jnp.float32),
            pltpu.SemaphoreType.DMA,
        ],
    )(tokens, dest_s)


def _sc_combine_body(eout_hbm, dst_hbm, out_hbm, idx_v, rows_v, sem):
    base = _sc_worker_id() * _SC_TOK
    pltpu.sync_copy(dst_hbm.at[pl.ds(base, _SC_TOK)], idx_v)
    pltpu.async_copy(eout_hbm.at[idx_v], rows_v, sem).wait()
    pltpu.sync_copy(rows_v, out_hbm.at[pl.ds(base, _SC_TOK)])


def _sc_combine(eout2d, dest_g):
    mesh = plsc.VectorSubcoreMesh(core_axis_name="c", subcore_axis_name="s")
    return pl.kernel(
        _sc_combine_body,
        out_type=jax.ShapeDtypeStruct((T, D), jnp.float32),
        mesh=mesh,
        scratch_types=[
            pltpu.VMEM((_SC_TOK,), jnp.int32),
            pltpu.VMEM((_SC_TOK, D), jnp.float32),
            pltpu.SemaphoreType.DMA,
        ],
    )(eout2d, dest_g)


NFS = 2
DFH = DFF // NFS


def _ffn_kernel(disp_ref, w1_ref, b1_ref, w2_ref, b2_ref, out_ref):
    j = pl.program_id(1)
    h1 = jnp.dot(disp_ref[...], w1_ref[0], preferred_element_type=jnp.float32)
    h1 = h1 + b1_ref[0]
    h1 = 0.5 * h1 * (1.0 + lax.erf(h1 * 0.7071067811865476))
    part = jnp.dot(h1, w2_ref[0], preferred_element_type=jnp.float32)

    @pl.when(j == 0)
    def _():
        out_ref[0] = part + b2_ref[0]

    @pl.when(j > 0)
    def _():
        out_ref[0] += part


def _ffn_call(disp, w1, b1, w2, b2, interpret=False):
    return pl.pallas_call(
        _ffn_kernel,
        grid=(E, NFS),
        in_specs=[
            pl.BlockSpec((CAP, D), lambda e, j: (e, 0)),
            pl.BlockSpec((1, D, DFH), lambda e, j: (e, 0, j)),
            pl.BlockSpec((1, 1, DFH), lambda e, j: (e, 0, j)),
            pl.BlockSpec((1, DFH, D), lambda e, j: (e, j, 0)),
            pl.BlockSpec((1, 1, D), lambda e, j: (e, 0, 0)),
        ],
        out_specs=pl.BlockSpec((1, CAP, D), lambda e, j: (e, 0, 0)),
        out_shape=jax.ShapeDtypeStruct((E, CAP, D), jnp.float32),
        interpret=interpret,
    )(disp, w1, b1.reshape(E, 1, DFF), w2, b2.reshape(E, 1, D))


def _epi_kernel(x2_ref, gat_ref, gv_ref, o_ref):
    o_ref[...] = x2_ref[...] + gat_ref[...] * gv_ref[...]


def _epi_call(x2, gathered, gate_val, interpret=False):
    return pl.pallas_call(
        _epi_kernel,
        grid=(NTB,),
        in_specs=[
            pl.BlockSpec((TBLK, D), lambda i: (i, 0)),
            pl.BlockSpec((TBLK, D), lambda i: (i, 0)),
            pl.BlockSpec((TBLK, 1), lambda i: (i, 0)),
        ],
        out_specs=pl.BlockSpec((TBLK, D), lambda i: (i, 0)),
        out_shape=jax.ShapeDtypeStruct((T, D), jnp.float32),
        interpret=interpret,
    )(x2, gathered, gate_val)


def kernel(x, ln1_g, ln1_b, in_proj_w, in_proj_b, out_proj_w, out_proj_b,
           ln2_g, ln2_b, gate_w, w1, b1, w2, b2):
    x2d = x.reshape(T, D)
    qkv = _qkv_call(x2d, ln1_g, ln1_b, in_proj_w, in_proj_b)
    ao = _attn_call(qkv)
    x2, tokens, dest_s, dest_g, gate_val = _post_call(
        ao, x2d, out_proj_w, out_proj_b, ln2_g, ln2_b, gate_w)
    disp = _sc_dispatch(tokens, dest_s.reshape(T))
    eout = _ffn_call(disp, w1, b1, w2, b2)
    gathered = _sc_combine(eout.reshape(T, D), dest_g.reshape(T))
    out = _epi_call(x2, gathered, gate_val)
    return out.reshape(B, S, D)
